# baseline (speedup 1.0000x reference)
import functools

import jax
import jax.numpy as jnp
from jax import lax
from jax.experimental import pallas as pl
from jax.experimental.pallas import tpu as pltpu
from jax.experimental.pallas import tpu_sc as plsc

_NC, _NS, _L = 2, 16, 16
_NW = _NC * _NS


def _stage_in(updates, mask, n, hb):
    b, h, w, c = updates.shape
    nt = hb * w
    grid = (b, h // hb)

    def tin(u_ref, m_ref, v_ref, i_ref):
        u = u_ref[0].reshape(nt, c)
        m = m_ref[0].reshape(nt, c)
        v_ref[...] = u.T
        i_ref[...] = (m >> 6).T // 3

    return pl.pallas_call(
        tin,
        grid=grid,
        compiler_params=pltpu.CompilerParams(allow_input_fusion=[True, True]),
        in_specs=[
            pl.BlockSpec((1, hb, w, c), lambda i, j: (i, j, 0, 0)),
            pl.BlockSpec((1, hb, w, c), lambda i, j: (i, j, 0, 0)),
        ],
        out_specs=[
            pl.BlockSpec((c, nt), lambda i, j: (i, j)),
            pl.BlockSpec((c, nt), lambda i, j: (i, j)),
        ],
        out_shape=[
            jax.ShapeDtypeStruct((b * c, n), jnp.float32),
            jax.ShapeDtypeStruct((b * c, n), jnp.int32),
        ],
    )(updates, mask.astype(jnp.int32))


def _stage_out(out_t, b, c, hout, wout, hb2):
    def tout(t_ref, o_ref):
        o_ref[...] = jnp.swapaxes(t_ref[...], 0, 1)[None]

    res = pl.pallas_call(
        tout,
        grid=(b, hout // hb2),
        in_specs=[pl.BlockSpec((c, hb2, wout), lambda i, j: (i, j, 0))],
        out_specs=pl.BlockSpec((1, hb2, c, wout), lambda i, j: (i, j, 0, 0)),
        out_shape=jax.ShapeDtypeStruct((b, hout, c, wout), jnp.float32),
    )(out_t)
    return jnp.transpose(res, (0, 1, 3, 2))


def _unpool_planes(vals_t, idx_t, n, hout, wout):
    rows = vals_t.shape[0]
    assert rows % _NW == 0
    items = rows // _NW
    chunks = 7
    rch = hout // chunks
    wvr = wout // _L
    sun = 8
    assert rch % 8 == 0 and wout % _L == 0 and n % (_L * sun) == 0

    mesh = plsc.VectorSubcoreMesh(
        core_axis_name="c", subcore_axis_name="s",
        num_cores=_NC, num_subcores=_NS,
    )

    @functools.partial(
        pl.kernel,
        out_type=jax.ShapeDtypeStruct((rows, hout, wout), jnp.float32),
        mesh=mesh,
        compiler_params=pltpu.CompilerParams(needs_layout_passes=False),
        scratch_types=[
            pltpu.VMEM((n,), jnp.float32),
            pltpu.VMEM((n,), jnp.int32),
            pltpu.VMEM((n,), jnp.float32),
            pltpu.VMEM((n,), jnp.int32),
            pltpu.VMEM((hout, wout), jnp.float32),
            pltpu.SemaphoreType.DMA,
            pltpu.SemaphoreType.DMA,
            pltpu.SemaphoreType.DMA,
            pltpu.SemaphoreType.DMA,
        ],
    )
    def k(vals_hbm, idx_hbm, out_hbm,
          va, ia, vb, ib, acc, s_a, s_b, so0, so1):
        wid = lax.axis_index("s") * _NC + lax.axis_index("c")
        inbufs = ((va, ia, s_a), (vb, ib, s_b))

        def start_in(row, which):
            vbuf, ibuf, sem = inbufs[which]
            pltpu.make_async_copy(vals_hbm.at[row], vbuf, sem).start()
            pltpu.make_async_copy(idx_hbm.at[row], ibuf, sem).start()

        def wait_in(which):
            vbuf, ibuf, sem = inbufs[which]
            pltpu.make_async_copy(vals_hbm.at[0], vbuf, sem).wait()
            pltpu.make_async_copy(idx_hbm.at[0], ibuf, sem).wait()

        zv = jnp.zeros((_L,), jnp.float32)

        def zero_chunk(seg):
            def zb(i, c_):
                r = seg * rch + i
                for u in range(wvr):
                    acc[r, pl.ds(u * _L, _L)] = zv
                return c_

            lax.fori_loop(0, rch, zb, 0)

        def scatter_item(which):
            vbuf, ibuf, _ = inbufs[which]

            def sb(i, c_):
                base = i * (_L * sun)
                for u in range(sun):
                    off = base + u * _L
                    idx = ibuf[pl.ds(off, _L)]
                    x = (idx >> 5).astype(jnp.float32)
                    r = ((x + 0.5) * (1.0 / 7.0)).astype(jnp.int32)
                    cc = idx - r * wout
                    v = vbuf[pl.ds(off, _L)]
                    plsc.addupdate_scatter(acc, [r, cc], v)
                return c_

            lax.fori_loop(0, n // (_L * sun), sb, 0)

        def drain_item(row):
            sems = (so0, so1)
            cps = []
            for j in range(chunks):
                cp = pltpu.make_async_copy(
                    acc.at[pl.ds(j * rch, rch)],
                    out_hbm.at[row, pl.ds(j * rch, rch)],
                    sems[j % 2],
                )
                cp.start()
                if j >= 1:
                    cps[j - 1].wait()
                    zero_chunk(j - 1)
                cps.append(cp)
            cps[-1].wait()
            zero_chunk(chunks - 1)

        start_in(wid, 0)
        for seg in range(chunks):
            zero_chunk(seg)

        def body(i2, c_):
            row_a = (2 * i2) * _NW + wid
            row_b = row_a + _NW
            start_in(row_b, 1)
            wait_in(0)
            scatter_item(0)
            drain_item(row_a)

            @pl.when(i2 + 1 < items // 2)
            def _():
                start_in(row_b + _NW, 0)

            wait_in(1)
            scatter_item(1)
            drain_item(row_b)
            return c_

        lax.fori_loop(0, items // 2, body, 0)

    return k(vals_t, idx_t)


def kernel(updates, mask):
    b, h, w, c = updates.shape
    n = h * w
    hout, wout = 2 * h, 2 * w
    p = hout * wout
    assert c == 192
    vals_t, idx_t = _stage_in(updates, mask, n, hb=16)
    out_t = _unpool_planes(vals_t, idx_t, n, hout, wout)
    return _stage_out(out_t, b, c, hout, wout, hb2=8)


# speedup vs baseline: 1.4612x; 1.4612x over previous
import functools

import jax
import jax.numpy as jnp
from jax import lax
from jax.experimental import pallas as pl
from jax.experimental.pallas import tpu as pltpu
from jax.experimental.pallas import tpu_sc as plsc

_NC, _NS, _L = 2, 16, 16
_NW = _NC * _NS


def _stage_in(updates, mask, n, hb):
    b, h, w, c = updates.shape
    nt = hb * w
    grid = (b, h // hb)

    def tin(u_ref, m_ref, v_ref, i_ref):
        u = u_ref[0].reshape(nt, c)
        m = m_ref[0].reshape(nt, c)
        v_ref[...] = u.T
        i_ref[...] = (m >> 6).T // 3

    return pl.pallas_call(
        tin,
        grid=grid,
        compiler_params=pltpu.CompilerParams(allow_input_fusion=[True, True]),
        in_specs=[
            pl.BlockSpec((1, hb, w, c), lambda i, j: (i, j, 0, 0)),
            pl.BlockSpec((1, hb, w, c), lambda i, j: (i, j, 0, 0)),
        ],
        out_specs=[
            pl.BlockSpec((c, nt), lambda i, j: (i, j)),
            pl.BlockSpec((c, nt), lambda i, j: (i, j)),
        ],
        out_shape=[
            jax.ShapeDtypeStruct((b * c, n), jnp.float32),
            jax.ShapeDtypeStruct((b * c, n), jnp.int32),
        ],
    )(updates, mask.astype(jnp.int32))


def _stage_out(out_t, b, c, hout, wout, pt):
    p = hout * wout
    rows_per_blk = pt // wout

    def tout(t_ref, o_ref):
        x = t_ref[...].reshape(c, rows_per_blk, wout)
        o_ref[...] = jnp.swapaxes(x, 0, 1)[None]

    res = pl.pallas_call(
        tout,
        grid=(b, p // pt),
        in_specs=[pl.BlockSpec((c, pt), lambda i, j: (i, j))],
        out_specs=pl.BlockSpec(
            (1, rows_per_blk, c, wout), lambda i, j: (i, j, 0, 0)
        ),
        out_shape=jax.ShapeDtypeStruct((b, hout, c, wout), jnp.float32),
    )(out_t)
    return jnp.transpose(res, (0, 1, 3, 2))


def _unpool_planes(vals_t, idx_t, n, p):
    rows = vals_t.shape[0]
    assert rows % _NW == 0
    items = rows // _NW
    chunks = 8
    ch = p // chunks
    zun = 8
    sun = 8
    assert ch % (_L * zun) == 0 and n % (_L * sun) == 0

    mesh = plsc.VectorSubcoreMesh(
        core_axis_name="c", subcore_axis_name="s",
        num_cores=_NC, num_subcores=_NS,
    )

    @functools.partial(
        pl.kernel,
        out_type=jax.ShapeDtypeStruct((rows, p), jnp.float32),
        mesh=mesh,
        compiler_params=pltpu.CompilerParams(needs_layout_passes=False),
        scratch_types=[
            pltpu.VMEM((n,), jnp.float32),
            pltpu.VMEM((n,), jnp.int32),
            pltpu.VMEM((n,), jnp.float32),
            pltpu.VMEM((n,), jnp.int32),
            pltpu.VMEM((p,), jnp.float32),
            pltpu.SemaphoreType.DMA,
            pltpu.SemaphoreType.DMA,
            pltpu.SemaphoreType.DMA,
            pltpu.SemaphoreType.DMA,
        ],
    )
    def k(vals_hbm, idx_hbm, out_hbm,
          va, ia, vb, ib, acc, s_a, s_b, so0, so1):
        wid = lax.axis_index("s") * _NC + lax.axis_index("c")
        inbufs = ((va, ia, s_a), (vb, ib, s_b))

        def start_in(row, which):
            vbuf, ibuf, sem = inbufs[which]
            pltpu.make_async_copy(vals_hbm.at[row], vbuf, sem).start()
            pltpu.make_async_copy(idx_hbm.at[row], ibuf, sem).start()

        def wait_in(which):
            vbuf, ibuf, sem = inbufs[which]
            pltpu.make_async_copy(vals_hbm.at[0], vbuf, sem).wait()
            pltpu.make_async_copy(idx_hbm.at[0], ibuf, sem).wait()

        zv = jnp.zeros((_L,), jnp.float32)

        def zero_chunk(seg):
            def zb(i, c_):
                off = seg * ch + i * (_L * zun)
                for u in range(zun):
                    acc[pl.ds(off + u * _L, _L)] = zv
                return c_

            lax.fori_loop(0, ch // (_L * zun), zb, 0)

        def scatter_item(which):
            vbuf, ibuf, _ = inbufs[which]

            def sb(i, c_):
                base = i * (_L * sun)
                for u in range(sun):
                    off = base + u * _L
                    idx = ibuf[pl.ds(off, _L)]
                    v = vbuf[pl.ds(off, _L)]
                    plsc.addupdate_scatter(acc, [idx], v)
                return c_

            lax.fori_loop(0, n // (_L * sun), sb, 0)

        def drain_item(row):
            sems = (so0, so1)
            cps = []
            for j in range(chunks):
                cp = pltpu.make_async_copy(
                    acc.at[pl.ds(j * ch, ch)],
                    out_hbm.at[row, pl.ds(j * ch, ch)],
                    sems[j % 2],
                )
                cp.start()
                if j >= 1:
                    cps[j - 1].wait()
                    zero_chunk(j - 1)
                cps.append(cp)
            cps[-1].wait()
            zero_chunk(chunks - 1)

        start_in(wid, 0)
        for seg in range(chunks):
            zero_chunk(seg)

        def body(i2, c_):
            row_a = (2 * i2) * _NW + wid
            row_b = row_a + _NW
            start_in(row_b, 1)
            wait_in(0)
            scatter_item(0)
            drain_item(row_a)

            @pl.when(i2 + 1 < items // 2)
            def _():
                start_in(row_b + _NW, 0)

            wait_in(1)
            scatter_item(1)
            drain_item(row_b)
            return c_

        lax.fori_loop(0, items // 2, body, 0)

    return k(vals_t, idx_t)


def kernel(updates, mask):
    b, h, w, c = updates.shape
    n = h * w
    hout, wout = 2 * h, 2 * w
    p = hout * wout
    assert c == 192
    vals_t, idx_t = _stage_in(updates, mask, n, hb=16)
    out_t = _unpool_planes(vals_t, idx_t, n, p)
    return _stage_out(out_t, b, c, hout, wout, pt=1792)


# speedup vs baseline: 1.7580x; 1.2032x over previous
import functools

import jax
import jax.numpy as jnp
from jax import lax
from jax.experimental import pallas as pl
from jax.experimental.pallas import tpu as pltpu
from jax.experimental.pallas import tpu_sc as plsc

_NC, _NS, _L = 2, 16, 16
_NW = _NC * _NS


def _stage_in(updates, mask, n, hb):
    b, h, w, c = updates.shape
    nt = hb * w
    grid = (b, h // hb)

    def tin(u_ref, m_ref, v_ref, i_ref):
        u = jnp.swapaxes(u_ref[0], 0, 1).reshape(c, nt)
        m = jnp.swapaxes(m_ref[0], 0, 1).reshape(c, nt)
        v_ref[...] = u
        i_ref[...] = (m >> 6) // 3

    return pl.pallas_call(
        tin,
        grid=grid,
        in_specs=[
            pl.BlockSpec((1, hb, c, w), lambda i, j: (i, j, 0, 0)),
            pl.BlockSpec((1, hb, c, w), lambda i, j: (i, j, 0, 0)),
        ],
        out_specs=[
            pl.BlockSpec((c, nt), lambda i, j: (i, j)),
            pl.BlockSpec((c, nt), lambda i, j: (i, j)),
        ],
        out_shape=[
            jax.ShapeDtypeStruct((b * c, n), jnp.float32),
            jax.ShapeDtypeStruct((b * c, n), jnp.int32),
        ],
    )(
        jnp.transpose(updates, (0, 1, 3, 2)),
        jnp.transpose(mask.astype(jnp.int32), (0, 1, 3, 2)),
    )


def _stage_out(out_t, b, c, hout, wout, pt):
    p = hout * wout
    rows_per_blk = pt // wout

    def tout(t_ref, o_ref):
        x = t_ref[...].reshape(c, rows_per_blk, wout)
        o_ref[...] = jnp.swapaxes(x, 0, 1)[None]

    res = pl.pallas_call(
        tout,
        grid=(b, p // pt),
        in_specs=[pl.BlockSpec((c, pt), lambda i, j: (i, j))],
        out_specs=pl.BlockSpec(
            (1, rows_per_blk, c, wout), lambda i, j: (i, j, 0, 0)
        ),
        out_shape=jax.ShapeDtypeStruct((b, hout, c, wout), jnp.float32),
    )(out_t)
    return jnp.transpose(res, (0, 1, 3, 2))


def _unpool_planes(vals_t, idx_t, n, p):
    rows = vals_t.shape[0]
    assert rows % _NW == 0
    items = rows // _NW
    chunks = 8
    ch = p // chunks
    zun = 8
    sun = 8
    assert ch % (_L * zun) == 0 and n % (_L * sun) == 0

    mesh = plsc.VectorSubcoreMesh(
        core_axis_name="c", subcore_axis_name="s",
        num_cores=_NC, num_subcores=_NS,
    )

    @functools.partial(
        pl.kernel,
        out_type=jax.ShapeDtypeStruct((rows, p), jnp.float32),
        mesh=mesh,
        compiler_params=pltpu.CompilerParams(needs_layout_passes=False),
        scratch_types=[
            pltpu.VMEM((n,), jnp.float32),
            pltpu.VMEM((n,), jnp.int32),
            pltpu.VMEM((n,), jnp.float32),
            pltpu.VMEM((n,), jnp.int32),
            pltpu.VMEM((p,), jnp.float32),
            pltpu.SemaphoreType.DMA,
            pltpu.SemaphoreType.DMA,
            pltpu.SemaphoreType.DMA,
            pltpu.SemaphoreType.DMA,
        ],
    )
    def k(vals_hbm, idx_hbm, out_hbm,
          va, ia, vb, ib, acc, s_a, s_b, so0, so1):
        wid = lax.axis_index("s") * _NC + lax.axis_index("c")
        inbufs = ((va, ia, s_a), (vb, ib, s_b))

        def start_in(row, which):
            vbuf, ibuf, sem = inbufs[which]
            pltpu.make_async_copy(vals_hbm.at[row], vbuf, sem).start()
            pltpu.make_async_copy(idx_hbm.at[row], ibuf, sem).start()

        def wait_in(which):
            vbuf, ibuf, sem = inbufs[which]
            pltpu.make_async_copy(vals_hbm.at[0], vbuf, sem).wait()
            pltpu.make_async_copy(idx_hbm.at[0], ibuf, sem).wait()

        zv = jnp.zeros((_L,), jnp.float32)

        def zero_chunk(seg):
            def zb(i, c_):
                off = seg * ch + i * (_L * zun)
                for u in range(zun):
                    acc[pl.ds(off + u * _L, _L)] = zv
                return c_

            lax.fori_loop(0, ch // (_L * zun), zb, 0)

        def scatter_item(which):
            vbuf, ibuf, _ = inbufs[which]

            def sb(i, c_):
                base = i * (_L * sun)
                for u in range(sun):
                    off = base + u * _L
                    idx = ibuf[pl.ds(off, _L)]
                    v = vbuf[pl.ds(off, _L)]
                    plsc.addupdate_scatter(acc, [idx], v)
                return c_

            lax.fori_loop(0, n // (_L * sun), sb, 0)

        def drain_item(row):
            sems = (so0, so1)
            cps = []
            for j in range(chunks):
                cp = pltpu.make_async_copy(
                    acc.at[pl.ds(j * ch, ch)],
                    out_hbm.at[row, pl.ds(j * ch, ch)],
                    sems[j % 2],
                )
                cp.start()
                if j >= 1:
                    cps[j - 1].wait()
                    zero_chunk(j - 1)
                cps.append(cp)
            cps[-1].wait()
            zero_chunk(chunks - 1)

        start_in(wid, 0)
        for seg in range(chunks):
            zero_chunk(seg)

        def body(i2, c_):
            row_a = (2 * i2) * _NW + wid
            row_b = row_a + _NW
            start_in(row_b, 1)
            wait_in(0)
            scatter_item(0)
            drain_item(row_a)

            @pl.when(i2 + 1 < items // 2)
            def _():
                start_in(row_b + _NW, 0)

            wait_in(1)
            scatter_item(1)
            drain_item(row_b)
            return c_

        lax.fori_loop(0, items // 2, body, 0)

    return k(vals_t, idx_t)


def kernel(updates, mask):
    b, h, w, c = updates.shape
    n = h * w
    hout, wout = 2 * h, 2 * w
    p = hout * wout
    assert c == 192
    vals_t, idx_t = _stage_in(updates, mask, n, hb=16)
    out_t = _unpool_planes(vals_t, idx_t, n, p)
    return _stage_out(out_t, b, c, hout, wout, pt=1792)


# speedup vs baseline: 2.2652x; 1.2885x over previous
import functools

import jax
import jax.numpy as jnp
from jax import lax
from jax.experimental import pallas as pl
from jax.experimental.pallas import tpu as pltpu
from jax.experimental.pallas import tpu_sc as plsc

_NC, _NS, _L = 2, 16, 16
_NW = _NC * _NS


def _stage_in(updates, mask, n, hb, b0, bg):
    b, h, c, w = updates.shape
    nt = hb * w
    grid = (bg, h // hb)

    def tin(u_ref, m_ref, v_ref, i_ref):
        u = jnp.swapaxes(u_ref[0], 0, 1).reshape(c, nt)
        m = jnp.swapaxes(m_ref[0], 0, 1).reshape(c, nt)
        v_ref[...] = u
        i_ref[...] = (m >> 6) // 3

    return pl.pallas_call(
        tin,
        grid=grid,
        in_specs=[
            pl.BlockSpec((1, hb, c, w), lambda i, j: (i + b0, j, 0, 0)),
            pl.BlockSpec((1, hb, c, w), lambda i, j: (i + b0, j, 0, 0)),
        ],
        out_specs=[
            pl.BlockSpec((c, nt), lambda i, j: (i, j)),
            pl.BlockSpec((c, nt), lambda i, j: (i, j)),
        ],
        out_shape=[
            jax.ShapeDtypeStruct((bg * c, n), jnp.float32),
            jax.ShapeDtypeStruct((bg * c, n), jnp.int32),
        ],
    )(updates, mask)


def _stage_out(out_t, prev, b, b0, bg, c, hout, wout, pt):
    p = hout * wout
    rows_per_blk = pt // wout
    out_shape = jax.ShapeDtypeStruct((b, hout, c, wout), jnp.float32)
    out_spec = pl.BlockSpec(
        (1, rows_per_blk, c, wout), lambda i, j: (i + b0, j, 0, 0)
    )
    grid = (bg, p // pt)

    def xform(t_ref):
        x = t_ref[...].reshape(c, rows_per_blk, wout)
        return jnp.swapaxes(x, 0, 1)[None]

    if prev is None:
        def tout(t_ref, o_ref):
            o_ref[...] = xform(t_ref)

        return pl.pallas_call(
            tout, grid=grid,
            in_specs=[pl.BlockSpec((c, pt), lambda i, j: (i, j))],
            out_specs=out_spec, out_shape=out_shape,
        )(out_t)

    def tout2(p_ref, t_ref, o_ref):
        del p_ref
        o_ref[...] = xform(t_ref)

    return pl.pallas_call(
        tout2, grid=grid,
        in_specs=[
            pl.BlockSpec(memory_space=pl.ANY),
            pl.BlockSpec((c, pt), lambda i, j: (i, j)),
        ],
        out_specs=out_spec, out_shape=out_shape,
        input_output_aliases={0: 0},
    )(prev, out_t)


def _unpool_planes(vals_t, idx_t, n, p):
    rows = vals_t.shape[0]
    assert rows % _NW == 0
    items = rows // _NW
    chunks = 8
    ch = p // chunks
    zun = 8
    sun = 8
    assert ch % (_L * zun) == 0 and n % (_L * sun) == 0

    mesh = plsc.VectorSubcoreMesh(
        core_axis_name="c", subcore_axis_name="s",
        num_cores=_NC, num_subcores=_NS,
    )

    @functools.partial(
        pl.kernel,
        out_type=jax.ShapeDtypeStruct((rows, p), jnp.float32),
        mesh=mesh,
        compiler_params=pltpu.CompilerParams(needs_layout_passes=False),
        scratch_types=[
            pltpu.VMEM((n,), jnp.float32),
            pltpu.VMEM((n,), jnp.int32),
            pltpu.VMEM((n,), jnp.float32),
            pltpu.VMEM((n,), jnp.int32),
            pltpu.VMEM((p,), jnp.float32),
            pltpu.SemaphoreType.DMA,
            pltpu.SemaphoreType.DMA,
            pltpu.SemaphoreType.DMA,
            pltpu.SemaphoreType.DMA,
        ],
    )
    def k(vals_hbm, idx_hbm, out_hbm,
          va, ia, vb, ib, acc, s_a, s_b, so0, so1):
        wid = lax.axis_index("s") * _NC + lax.axis_index("c")
        inbufs = ((va, ia, s_a), (vb, ib, s_b))

        def start_in(row, which):
            vbuf, ibuf, sem = inbufs[which]
            pltpu.make_async_copy(vals_hbm.at[row], vbuf, sem).start()
            pltpu.make_async_copy(idx_hbm.at[row], ibuf, sem).start()

        def wait_in(which):
            vbuf, ibuf, sem = inbufs[which]
            pltpu.make_async_copy(vals_hbm.at[0], vbuf, sem).wait()
            pltpu.make_async_copy(idx_hbm.at[0], ibuf, sem).wait()

        zv = jnp.zeros((_L,), jnp.float32)

        def zero_chunk(seg):
            def zb(i, c_):
                off = seg * ch + i * (_L * zun)
                for u in range(zun):
                    acc[pl.ds(off + u * _L, _L)] = zv
                return c_

            lax.fori_loop(0, ch // (_L * zun), zb, 0)

        def scatter_item(which):
            vbuf, ibuf, _ = inbufs[which]

            def sb(i, c_):
                base = i * (_L * sun)
                for u in range(sun):
                    off = base + u * _L
                    idx = ibuf[pl.ds(off, _L)]
                    v = vbuf[pl.ds(off, _L)]
                    plsc.addupdate_scatter(acc, [idx], v)
                return c_

            lax.fori_loop(0, n // (_L * sun), sb, 0)

        def drain_item(row):
            sems = (so0, so1)
            cps = []
            for j in range(chunks):
                cp = pltpu.make_async_copy(
                    acc.at[pl.ds(j * ch, ch)],
                    out_hbm.at[row, pl.ds(j * ch, ch)],
                    sems[j % 2],
                )
                cp.start()
                if j >= 1:
                    cps[j - 1].wait()
                    zero_chunk(j - 1)
                cps.append(cp)
            cps[-1].wait()
            zero_chunk(chunks - 1)

        start_in(wid, 0)
        for seg in range(chunks):
            zero_chunk(seg)

        def body(i2, c_):
            row_a = (2 * i2) * _NW + wid
            row_b = row_a + _NW
            start_in(row_b, 1)
            wait_in(0)
            scatter_item(0)
            drain_item(row_a)

            @pl.when(i2 + 1 < items // 2)
            def _():
                start_in(row_b + _NW, 0)

            wait_in(1)
            scatter_item(1)
            drain_item(row_b)
            return c_

        lax.fori_loop(0, items // 2, body, 0)

    return k(vals_t, idx_t)


def kernel(updates, mask):
    b, h, w, c = updates.shape
    n = h * w
    hout, wout = 2 * h, 2 * w
    p = hout * wout
    assert c == 192
    u_v = jnp.transpose(updates, (0, 1, 3, 2))
    m_v = jnp.transpose(mask.astype(jnp.int32), (0, 1, 3, 2))
    groups = 2
    bg = b // groups
    res = None
    for g in range(groups):
        vals_t, idx_t = _stage_in(u_v, m_v, n, hb=16, b0=g * bg, bg=bg)
        out_t = _unpool_planes(vals_t, idx_t, n, p)
        res = _stage_out(out_t, res, b, g * bg, bg, c, hout, wout, pt=1792)
    return jnp.transpose(res, (0, 1, 3, 2))


# speedup vs baseline: 2.4826x; 1.0960x over previous
import functools

import jax
import jax.numpy as jnp
from jax import lax
from jax.experimental import pallas as pl
from jax.experimental.pallas import tpu as pltpu
from jax.experimental.pallas import tpu_sc as plsc

_NC, _NS, _L = 2, 16, 16
_NW = _NC * _NS


def _stage_in(updates, mask, n, hb, b0, bg):
    b, h, c, w = updates.shape
    nt = hb * w
    grid = (bg, h // hb)

    def tin(u_ref, m_ref, v_ref, i_ref):
        u = jnp.swapaxes(u_ref[0], 0, 1).reshape(c, nt)
        m = jnp.swapaxes(m_ref[0], 0, 1).reshape(c, nt)
        v_ref[...] = u
        i_ref[...] = (m >> 6) // 3

    return pl.pallas_call(
        tin,
        grid=grid,
        in_specs=[
            pl.BlockSpec((1, hb, c, w), lambda i, j: (i + b0, j, 0, 0)),
            pl.BlockSpec((1, hb, c, w), lambda i, j: (i + b0, j, 0, 0)),
        ],
        out_specs=[
            pl.BlockSpec((c, nt), lambda i, j: (i, j)),
            pl.BlockSpec((c, nt), lambda i, j: (i, j)),
        ],
        out_shape=[
            jax.ShapeDtypeStruct((bg * c, n), jnp.float32),
            jax.ShapeDtypeStruct((bg * c, n), jnp.int32),
        ],
    )(updates, mask)


def _stage_out(out_t, prev, b, b0, bg, c, hout, wout, pt):
    p = hout * wout
    rows_per_blk = pt // wout
    out_shape = jax.ShapeDtypeStruct((b, hout, c, wout), jnp.float32)
    out_spec = pl.BlockSpec(
        (1, rows_per_blk, c, wout), lambda i, j: (i + b0, j, 0, 0)
    )
    grid = (bg, p // pt)

    def xform(t_ref):
        x = t_ref[...].reshape(c, rows_per_blk, wout)
        return jnp.swapaxes(x, 0, 1)[None]

    if prev is None:
        def tout(t_ref, o_ref):
            o_ref[...] = xform(t_ref)

        return pl.pallas_call(
            tout, grid=grid,
            in_specs=[pl.BlockSpec((c, pt), lambda i, j: (i, j))],
            out_specs=out_spec, out_shape=out_shape,
        )(out_t)

    def tout2(p_ref, t_ref, o_ref):
        del p_ref
        o_ref[...] = xform(t_ref)

    return pl.pallas_call(
        tout2, grid=grid,
        in_specs=[
            pl.BlockSpec(memory_space=pl.ANY),
            pl.BlockSpec((c, pt), lambda i, j: (i, j)),
        ],
        out_specs=out_spec, out_shape=out_shape,
        input_output_aliases={0: 0},
    )(prev, out_t)


def _unpool_planes(vals_t, idx_t, n, p):
    rows = vals_t.shape[0]
    assert rows % _NW == 0
    items = rows // _NW
    chunks = 8
    ch = p // chunks
    zun = 8
    sun = 8
    assert ch % (_L * zun) == 0 and n % (_L * sun) == 0

    mesh = plsc.VectorSubcoreMesh(
        core_axis_name="c", subcore_axis_name="s",
        num_cores=_NC, num_subcores=_NS,
    )

    @functools.partial(
        pl.kernel,
        out_type=jax.ShapeDtypeStruct((rows, p), jnp.float32),
        mesh=mesh,
        compiler_params=pltpu.CompilerParams(needs_layout_passes=False),
        scratch_types=[
            pltpu.VMEM((n,), jnp.float32),
            pltpu.VMEM((n,), jnp.int32),
            pltpu.VMEM((n,), jnp.float32),
            pltpu.VMEM((n,), jnp.int32),
            pltpu.VMEM((p,), jnp.float32),
            pltpu.SemaphoreType.DMA,
            pltpu.SemaphoreType.DMA,
            pltpu.SemaphoreType.DMA,
            pltpu.SemaphoreType.DMA,
        ],
    )
    def k(vals_hbm, idx_hbm, out_hbm,
          va, ia, vb, ib, acc, s_a, s_b, so0, so1):
        wid = lax.axis_index("s") * _NC + lax.axis_index("c")
        inbufs = ((va, ia, s_a), (vb, ib, s_b))

        def start_in(row, which):
            vbuf, ibuf, sem = inbufs[which]
            pltpu.make_async_copy(vals_hbm.at[row], vbuf, sem).start()
            pltpu.make_async_copy(idx_hbm.at[row], ibuf, sem).start()

        def wait_in(which):
            vbuf, ibuf, sem = inbufs[which]
            pltpu.make_async_copy(vals_hbm.at[0], vbuf, sem).wait()
            pltpu.make_async_copy(idx_hbm.at[0], ibuf, sem).wait()

        zv = jnp.zeros((_L,), jnp.float32)

        def zero_chunk(seg):
            def zb(i, c_):
                off = seg * ch + i * (_L * zun)
                for u in range(zun):
                    acc[pl.ds(off + u * _L, _L)] = zv
                return c_

            lax.fori_loop(0, ch // (_L * zun), zb, 0)

        def scatter_item(which):
            vbuf, ibuf, _ = inbufs[which]

            def sb(i, c_):
                base = i * (_L * sun)
                for u in range(sun):
                    off = base + u * _L
                    idx = ibuf[pl.ds(off, _L)]
                    v = vbuf[pl.ds(off, _L)]
                    plsc.addupdate_scatter(acc, [idx], v)
                return c_

            lax.fori_loop(0, n // (_L * sun), sb, 0)

        def drain_item(row):
            sems = (so0, so1)
            cps = []
            for j in range(chunks):
                cp = pltpu.make_async_copy(
                    acc.at[pl.ds(j * ch, ch)],
                    out_hbm.at[row, pl.ds(j * ch, ch)],
                    sems[j % 2],
                )
                cp.start()
                if j >= 1:
                    cps[j - 1].wait()
                    zero_chunk(j - 1)
                cps.append(cp)
            cps[-1].wait()
            zero_chunk(chunks - 1)

        start_in(wid, 0)
        for seg in range(chunks):
            zero_chunk(seg)

        def body(i2, c_):
            row_a = (2 * i2) * _NW + wid
            row_b = row_a + _NW
            start_in(row_b, 1)
            wait_in(0)
            scatter_item(0)
            drain_item(row_a)

            @pl.when(i2 + 1 < items // 2)
            def _():
                start_in(row_b + _NW, 0)

            wait_in(1)
            scatter_item(1)
            drain_item(row_b)
            return c_

        lax.fori_loop(0, items // 2, body, 0)

    return k(vals_t, idx_t)


def kernel(updates, mask):
    b, h, w, c = updates.shape
    n = h * w
    hout, wout = 2 * h, 2 * w
    p = hout * wout
    assert c == 192
    u_v = jnp.transpose(updates, (0, 1, 3, 2))
    m_v = jnp.transpose(mask.astype(jnp.int32), (0, 1, 3, 2))
    groups = 4
    bg = b // groups
    res = None
    for g in range(groups):
        vals_t, idx_t = _stage_in(u_v, m_v, n, hb=16, b0=g * bg, bg=bg)
        out_t = _unpool_planes(vals_t, idx_t, n, p)
        res = _stage_out(out_t, res, b, g * bg, bg, c, hout, wout, pt=1792)
    return jnp.transpose(res, (0, 1, 3, 2))
